# Initial kernel scaffold; baseline (speedup 1.0000x reference)
import jax
import jax.numpy as jnp
from jax.experimental import pallas as pl


def kernel(coords, fields, multiscale_csr, W_enc, b_enc, Wq, Wk, Wv, Wo, bo, tau):
    raise NotImplementedError("write your pallas kernel here")


# speedup vs baseline: 1.7572x; 1.7572x over previous
import numpy as np
import jax
import jax.numpy as jnp
from jax.experimental import pallas as pl
from jax.experimental.pallas import tpu as pltpu

_N_BLK = 1000
_I0 = np.int32(0)


def _enc_body(xin_ref, w_ref, b_ref, o_ref):
    acc = jnp.dot(xin_ref[...], w_ref[...], preferred_element_type=jnp.float32)
    o_ref[...] = jax.nn.gelu(acc + b_ref[...])


def kernel(coords, fields, multiscale_csr, W_enc, b_enc, Wq, Wk, Wv, Wo, bo, tau):
    src = multiscale_csr[0]
    dst = multiscale_csr[1]
    n = coords.shape[0]
    rel = coords[src] - coords[dst]
    dist = jnp.sqrt(jnp.sum(rel * rel, axis=-1, keepdims=True) + 1e-12)
    ones = jnp.ones((src.shape[0],), jnp.float32)
    deg = jax.ops.segment_sum(ones, dst, num_segments=n)
    degc = jnp.maximum(deg, 1.0)[:, None]
    mean_rel = jax.ops.segment_sum(rel, dst, num_segments=n) / degc
    mean_dist = jax.ops.segment_sum(dist, dst, num_segments=n) / degc
    geom = jnp.concatenate([coords, mean_rel, mean_dist, jnp.log1p(deg)[:, None]], axis=-1)

    xin = jnp.concatenate([fields, geom], axis=-1).astype(jnp.float32)
    k_in = xin.shape[1]
    k_pad = 256
    xin = jnp.pad(xin, ((0, 0), (0, k_pad - k_in)))
    wp = jnp.pad(W_enc.astype(jnp.float32), ((0, k_pad - k_in), (0, 0)))
    b2 = b_enc.astype(jnp.float32).reshape(1, -1)

    x = pl.pallas_call(
        _enc_body,
        grid=(n // _N_BLK,),
        in_specs=[
            pl.BlockSpec((_N_BLK, k_pad), lambda i: (i, _I0)),
            pl.BlockSpec((k_pad, 128), lambda i: (_I0, _I0)),
            pl.BlockSpec((1, 128), lambda i: (_I0, _I0)),
        ],
        out_specs=pl.BlockSpec((_N_BLK, 128), lambda i: (i, _I0)),
        out_shape=jax.ShapeDtypeStruct((n, 128), jnp.float32),
    )(xin, wp, b2)

    for l in range(Wq.shape[0]):
        q = x @ Wq[l]
        k = x @ Wk[l]
        v = x @ Wv[l]
        qn = q / (jnp.linalg.norm(q, axis=-1, keepdims=True) + 1e-6)
        kn = k / (jnp.linalg.norm(k, axis=-1, keepdims=True) + 1e-6)
        score = jnp.sum(qn[dst] * kn[src], axis=-1) * tau[l]
        a = jnp.exp(score)
        denom = jax.ops.segment_sum(a, dst, num_segments=n)
        num = jax.ops.segment_sum(a[:, None] * v[src], dst, num_segments=n)
        msg = num / (denom + 1e-9)[:, None]
        h = jnp.concatenate([x, msg, geom], axis=-1) @ Wo[l] + bo[l]
        x = x + jax.nn.gelu(h)
    return x


# speedup vs baseline: 4.5786x; 2.6056x over previous
import functools
import numpy as np
import jax
import jax.numpy as jnp
from jax import lax
from jax.experimental import pallas as pl
from jax.experimental.pallas import tpu as pltpu
from jax.experimental.pallas import tpu_sc as plsc

NC = 2
NS = 16
NW = NC * NS
C = 128
BLK = 1000

_I0 = np.int32(0)
f32 = jnp.float32
i32 = jnp.int32


def _rsqrt16(d):
    i = plsc.bitcast(d, i32)
    i = jnp.int32(0x5F3759DF) - lax.shift_right_arithmetic(i, jnp.int32(1))
    y = plsc.bitcast(i, f32)
    for _ in range(3):
        y = y * (jnp.float32(1.5) - jnp.float32(0.5) * d * y * y)
    return y


def _allsum16(p, li):
    for k in (8, 4, 2, 1):
        idx = lax.bitwise_xor(li, jnp.int32(k))
        perm = lax.gather(
            p, idx[:, None],
            lax.GatherDimensionNumbers(offset_dims=(),
                                       collapsed_slice_dims=(0,),
                                       start_index_map=(0,)),
            slice_sizes=(1,),
            mode=lax.GatherScatterMode.PROMISE_IN_BOUNDS)
        p = p + perm
    return p


def _worker_chunks(n_chunks):
    w = lax.axis_index("s") * NC + lax.axis_index("c")
    base = int(n_chunks) // NW
    extra = int(n_chunks) - base * NW
    nw_chunks = jnp.where(w < np.int32(extra),
                          np.int32(base + 1), np.int32(base))
    return w.astype(i32), nw_chunks


def _geom_body(n, n_chunks, c16_hbm, src_hbm, dst_hbm, out_hbm,
               sidx, didx, srows, drows, stag, acc, sem):
    cid = lax.axis_index("c").astype(i32)
    sid = lax.axis_index("s").astype(i32)
    w, nw_chunks = _worker_chunks(n_chunks)
    rows_per_tile = n // NS

    li = lax.iota(i32, 16)
    lt2 = li < 2
    eq2 = li == 2
    one3 = jnp.where(li == 3, jnp.float32(1.0), jnp.float32(0.0))
    z16 = jnp.zeros((16,), f32)

    def zloop(_, r):
        stag[r, pl.ds(0, 16)] = z16
        return r + np.int32(1)
    lax.fori_loop(np.int32(0), np.int32(C), zloop, jnp.int32(0), unroll=8)
    for t in range(rows_per_tile // C):
        pltpu.sync_copy(stag.at[pl.ds(0, C)],
                        acc.at[pl.ds(sid * rows_per_tile + t * C, C)])
    rem = rows_per_tile % C
    if rem:
        pltpu.sync_copy(stag.at[pl.ds(0, rem)],
                        acc.at[pl.ds(sid * rows_per_tile
                                     + (rows_per_tile // C) * C, rem)])
    plsc.subcore_barrier()

    def chunk_body(j, _):
        base = (w + NW * j) * C
        pltpu.sync_copy(src_hbm.at[pl.ds(base, C)], sidx)
        pltpu.sync_copy(dst_hbm.at[pl.ds(base, C)], didx)
        cp1 = pltpu.async_copy(c16_hbm.at[sidx], srows, sem)
        cp2 = pltpu.async_copy(c16_hbm.at[didx], drows, sem)
        cp1.wait()
        cp2.wait()

        def edge_body(_, e):
            rel = srows[e, pl.ds(0, 16)] - drows[e, pl.ds(0, 16)]
            sq = rel * rel
            dv = _allsum16(sq, li) + jnp.float32(1e-12)
            distv = dv * _rsqrt16(dv)
            row = (jnp.where(lt2, rel, jnp.float32(0.0))
                   + jnp.where(eq2, distv, jnp.float32(0.0)) + one3)
            stag[e, pl.ds(0, 16)] = row
            return e + np.int32(1)
        lax.fori_loop(np.int32(0), np.int32(C), edge_body, jnp.int32(0),
                      unroll=4)
        pltpu.sync_copy(stag, acc.at[didx], add=True)
        return 0
    lax.fori_loop(jnp.int32(0), nw_chunks, chunk_body, 0)

    plsc.subcore_barrier()

    @pl.when(sid == 0)
    def _():
        pltpu.sync_copy(acc, out_hbm.at[cid])


def _geom_sc(coords16, src32, dst32, n, e):
    n_chunks = e // C
    mesh = plsc.VectorSubcoreMesh(core_axis_name="c", subcore_axis_name="s")
    kern = functools.partial(
        pl.kernel,
        out_type=jax.ShapeDtypeStruct((NC, n, 16), f32),
        mesh=mesh,
        compiler_params=pltpu.CompilerParams(needs_layout_passes=False, use_tc_tiling_on_sc=False),
        scratch_types=[
            pltpu.VMEM((C,), i32),
            pltpu.VMEM((C,), i32),
            pltpu.VMEM((C, 16), f32),
            pltpu.VMEM((C, 16), f32),
            pltpu.VMEM((C, 16), f32),
            pltpu.VMEM_SHARED((n, 16), f32),
            pltpu.SemaphoreType.DMA,
        ],
    )(functools.partial(_geom_body, n, n_chunks))
    return kern(coords16, src32, dst32)


def _attn_body(n, n_chunks, qnt_hbm, knn_hbm, xlo_hbm, xhi_hbm,
               src_hbm, dst_hbm, num_hbm, den_hbm,
               sidx, didx, qrows, krows, xrows, nstag, dstag,
               accn, accd, sem):
    cid = lax.axis_index("c").astype(i32)
    sid = lax.axis_index("s").astype(i32)
    base_c = int(n_chunks) // NS
    extra = int(n_chunks) - base_c * NS
    nt_chunks = jnp.where(sid < np.int32(extra),
                          np.int32(base_c + 1), np.int32(base_c))
    rows_per_tile = n // NS

    li = lax.iota(i32, 16)
    oh0 = jnp.where(li == 0, jnp.float32(1.0), jnp.float32(0.0))
    z16 = jnp.zeros((16,), f32)

    def zloop(_, r):
        for cc in range(4):
            nstag[r, pl.ds(16 * cc, 16)] = z16
        dstag[r, pl.ds(0, 16)] = z16
        return r + np.int32(1)
    lax.fori_loop(np.int32(0), np.int32(C), zloop, jnp.int32(0), unroll=4)
    done = 0
    while done < rows_per_tile:
        step = min(C, rows_per_tile - done)
        off = sid * rows_per_tile + done
        pltpu.sync_copy(nstag.at[pl.ds(0, step)], accn.at[pl.ds(off, step)])
        pltpu.sync_copy(dstag.at[pl.ds(0, step)], accd.at[pl.ds(off, step)])
        done += step
    plsc.subcore_barrier()

    def chunk_body(j, _):
        base = (sid + NS * j) * C
        pltpu.sync_copy(src_hbm.at[pl.ds(base, C)], sidx)
        pltpu.sync_copy(dst_hbm.at[pl.ds(base, C)], didx)
        cp1 = pltpu.async_copy(qnt_hbm.at[didx], qrows, sem)
        cp2 = pltpu.async_copy(knn_hbm.at[sidx], krows, sem)
        cp1.wait()
        cp2.wait()

        @pl.when(cid == 0)
        def _():
            pltpu.async_copy(xlo_hbm.at[sidx], xrows, sem).wait()

        @pl.when(cid == 1)
        def _():
            pltpu.async_copy(xhi_hbm.at[sidx], xrows, sem).wait()

        def edge_body(_, e):
            p = qrows[e, pl.ds(0, 16)] * krows[e, pl.ds(0, 16)]
            for cc in range(1, 4):
                p = p + (qrows[e, pl.ds(16 * cc, 16)]
                         * krows[e, pl.ds(16 * cc, 16)])
            av = jnp.exp(_allsum16(p, li))
            for cc in range(4):
                nstag[e, pl.ds(16 * cc, 16)] = (
                    av * xrows[e, pl.ds(16 * cc, 16)])
            dstag[e, pl.ds(0, 16)] = av * oh0
            return e + np.int32(1)
        lax.fori_loop(np.int32(0), np.int32(C), edge_body, jnp.int32(0),
                      unroll=2)
        pltpu.sync_copy(nstag, accn.at[didx], add=True)
        pltpu.sync_copy(dstag, accd.at[didx], add=True)
        return 0
    lax.fori_loop(jnp.int32(0), nt_chunks, chunk_body, 0)

    plsc.subcore_barrier()

    @pl.when(sid == 0)
    def _():
        pltpu.sync_copy(accn, num_hbm.at[cid])
        pltpu.sync_copy(accd, den_hbm.at[cid])


def _attn_sc(qnt, knn, xlo, xhi, src32, dst32, n, e):
    n_chunks = e // C
    mesh = plsc.VectorSubcoreMesh(core_axis_name="c", subcore_axis_name="s")
    kern = functools.partial(
        pl.kernel,
        out_type=(jax.ShapeDtypeStruct((NC, n, 64), f32),
                  jax.ShapeDtypeStruct((NC, n, 16), f32)),
        mesh=mesh,
        compiler_params=pltpu.CompilerParams(needs_layout_passes=False,
                                             use_tc_tiling_on_sc=False),
        scratch_types=[
            pltpu.VMEM((C,), i32),
            pltpu.VMEM((C,), i32),
            pltpu.VMEM((C, 64), f32),
            pltpu.VMEM((C, 64), f32),
            pltpu.VMEM((C, 64), f32),
            pltpu.VMEM((C, 64), f32),
            pltpu.VMEM((C, 16), f32),
            pltpu.VMEM_SHARED((n, 64), f32),
            pltpu.VMEM_SHARED((n, 16), f32),
            pltpu.SemaphoreType.DMA,
        ],
    )(functools.partial(_attn_body, n, n_chunks))
    return kern(qnt, knn, xlo, xhi, src32, dst32)


def _enc_body(gacc_ref, c2_ref, f_ref, F_ref, G8_ref, b_ref, plo_ref, phi_ref,
              x_ref, xlo_ref, xhi_ref, geom8_ref):
    acc = gacc_ref[0] + gacc_ref[1]
    deg = acc[:, 3:4]
    degc = jnp.maximum(deg, jnp.float32(1.0))
    mr = acc[:, 0:2] / degc
    md = acc[:, 2:3] / degc
    ld = jnp.log1p(deg)
    cc = c2_ref[...]
    geom8 = jnp.concatenate(
        [cc, mr, md, ld, jnp.zeros_like(cc)], axis=1)
    geom8_ref[...] = geom8
    h = (jnp.dot(f_ref[...], F_ref[...], preferred_element_type=f32)
         + jnp.dot(geom8, G8_ref[...], preferred_element_type=f32)
         + b_ref[...])
    x = jax.nn.gelu(h)
    x_ref[...] = x
    xlo_ref[...] = jnp.dot(x, plo_ref[...], preferred_element_type=f32)
    xhi_ref[...] = jnp.dot(x, phi_ref[...], preferred_element_type=f32)


def _qk_body(x_ref, wq_ref, wk_ref, tau_ref, qnt_ref, knn_ref):
    x = x_ref[...]
    q = jnp.dot(x, wq_ref[...], preferred_element_type=f32)
    k = jnp.dot(x, wk_ref[...], preferred_element_type=f32)
    qn = q / (jnp.sqrt(jnp.sum(q * q, axis=1, keepdims=True))
              + jnp.float32(1e-6))
    kn = k / (jnp.sqrt(jnp.sum(k * k, axis=1, keepdims=True))
              + jnp.float32(1e-6))
    qnt_ref[...] = qn * tau_ref[...]
    knn_ref[...] = kn


def _out_body(num_ref, den_ref, x_ref, g8_ref, wv_ref, B_ref, A_ref,
              Cg_ref, bo_ref, plo_ref, phi_ref,
              xo_ref, xolo_ref, xohi_ref):
    dn = (den_ref[0][:, 0:1] + den_ref[1][:, 0:1]) * jnp.float32(0.5)
    rden = jnp.float32(1.0) / (dn + jnp.float32(1e-9))
    m0 = num_ref[0] * rden
    m1 = num_ref[1] * rden
    wvb = jnp.dot(wv_ref[...], B_ref[...], preferred_element_type=f32)
    x = x_ref[...]
    h = (jnp.dot(x, A_ref[...], preferred_element_type=f32)
         + jnp.dot(m0, wvb[0:64, :], preferred_element_type=f32)
         + jnp.dot(m1, wvb[64:128, :], preferred_element_type=f32)
         + jnp.dot(g8_ref[...], Cg_ref[...], preferred_element_type=f32)
         + bo_ref[...])
    xo = x + jax.nn.gelu(h)
    xo_ref[...] = xo
    xolo_ref[...] = jnp.dot(xo, plo_ref[...], preferred_element_type=f32)
    xohi_ref[...] = jnp.dot(xo, phi_ref[...], preferred_element_type=f32)


def _row_spec(width):
    return pl.BlockSpec((BLK, width), lambda idx: (idx, _I0))


def _full_spec(shape):
    nd = len(shape)
    return pl.BlockSpec(shape, lambda idx: (_I0,) * nd)


def _acc_spec(width):
    return pl.BlockSpec((NC, BLK, width), lambda idx: (_I0, idx, _I0))


def kernel(coords, fields, multiscale_csr, W_enc, b_enc, Wq, Wk, Wv, Wo, bo,
           tau):
    n = coords.shape[0]
    e = multiscale_csr.shape[1]
    layers = Wq.shape[0]
    grid = (n // BLK,)

    src32 = multiscale_csr[0].astype(i32)
    dst32 = multiscale_csr[1].astype(i32)
    coords16 = jnp.zeros((n, 16), f32).at[:, 0:2].set(coords.astype(f32))

    gacc = _geom_sc(coords16, src32, dst32, n, e)

    Fw = W_enc[:128].astype(f32)
    G8 = jnp.zeros((8, 128), f32).at[0:6, :].set(W_enc[128:134].astype(f32))
    b_row = b_enc.astype(f32).reshape(1, 128)
    eye = jnp.eye(128, dtype=f32)
    plo = eye[:, 0:64]
    phi = eye[:, 64:128]

    x, xlo, xhi, geom8 = pl.pallas_call(
        _enc_body,
        grid=grid,
        in_specs=[
            _acc_spec(16),
            _row_spec(2),
            _row_spec(128),
            _full_spec((128, 128)),
            _full_spec((8, 128)),
            _full_spec((1, 128)),
            _full_spec((128, 64)),
            _full_spec((128, 64)),
        ],
        out_specs=(_row_spec(128), _row_spec(64), _row_spec(64),
                   _row_spec(8)),
        out_shape=(jax.ShapeDtypeStruct((n, 128), f32),
                   jax.ShapeDtypeStruct((n, 64), f32),
                   jax.ShapeDtypeStruct((n, 64), f32),
                   jax.ShapeDtypeStruct((n, 8), f32)),
    )(gacc, coords.astype(f32), fields.astype(f32), Fw, G8, b_row, plo, phi)

    for l in range(layers):
        tau_row = jnp.full((1, 64), tau[l], f32)
        qnt, knn = pl.pallas_call(
            _qk_body,
            grid=grid,
            in_specs=[
                _row_spec(128),
                _full_spec((128, 64)),
                _full_spec((128, 64)),
                _full_spec((1, 64)),
            ],
            out_specs=(_row_spec(64), _row_spec(64)),
            out_shape=(jax.ShapeDtypeStruct((n, 64), f32),
                       jax.ShapeDtypeStruct((n, 64), f32)),
        )(x, Wq[l].astype(f32), Wk[l].astype(f32), tau_row)

        num, den = _attn_sc(qnt, knn, xlo, xhi, src32, dst32, n, e)

        x, xlo, xhi = pl.pallas_call(
            _out_body,
            grid=grid,
            in_specs=[
                _acc_spec(64),
                _acc_spec(16),
                _row_spec(128),
                _row_spec(8),
                _full_spec((128, 128)),
                _full_spec((128, 128)),
                _full_spec((128, 128)),
                _full_spec((8, 128)),
                _full_spec((1, 128)),
                _full_spec((128, 64)),
                _full_spec((128, 64)),
            ],
            out_specs=(_row_spec(128), _row_spec(64), _row_spec(64)),
            out_shape=(jax.ShapeDtypeStruct((n, 128), f32),
                       jax.ShapeDtypeStruct((n, 64), f32),
                       jax.ShapeDtypeStruct((n, 64), f32)),
        )(num, den, x, geom8, Wv[l].astype(f32),
          Wo[l, 128:256].astype(f32), Wo[l, 0:128].astype(f32),
          jnp.zeros((8, 128), f32).at[0:6, :].set(Wo[l, 256:262].astype(f32)),
          bo[l].astype(f32).reshape(1, 128), plo, phi)
    return x


# speedup vs baseline: 7.2739x; 1.5887x over previous
import functools
import numpy as np
import jax
import jax.numpy as jnp
from jax import lax
from jax.experimental import pallas as pl
from jax.experimental.pallas import tpu as pltpu
from jax.experimental.pallas import tpu_sc as plsc

NC = 2
NS = 16
NW = NC * NS
C = 80
BLK = 1000

_I0 = np.int32(0)
f32 = jnp.float32
i32 = jnp.int32


def _rsqrt16(d):
    i = plsc.bitcast(d, i32)
    i = jnp.int32(0x5F3759DF) - lax.shift_right_arithmetic(i, jnp.int32(1))
    y = plsc.bitcast(i, f32)
    for _ in range(3):
        y = y * (jnp.float32(1.5) - jnp.float32(0.5) * d * y * y)
    return y


def _allsum16(p, li):
    for k in (8, 4, 2, 1):
        idx = lax.bitwise_xor(li, jnp.int32(k))
        perm = lax.gather(
            p, idx[:, None],
            lax.GatherDimensionNumbers(offset_dims=(),
                                       collapsed_slice_dims=(0,),
                                       start_index_map=(0,)),
            slice_sizes=(1,),
            mode=lax.GatherScatterMode.PROMISE_IN_BOUNDS)
        p = p + perm
    return p


def _worker_chunks(n_chunks):
    w = lax.axis_index("s") * NC + lax.axis_index("c")
    base = int(n_chunks) // NW
    extra = int(n_chunks) - base * NW
    nw_chunks = jnp.where(w < np.int32(extra),
                          np.int32(base + 1), np.int32(base))
    return w.astype(i32), nw_chunks


def _geom_body(n, n_chunks, c16_hbm, src_hbm, dst_hbm, out_hbm,
               sidxs, didxs, srows, drows, stag, acc,
               semi, semg0, semg1, sems0, sems1):
    cid = lax.axis_index("c").astype(i32)
    sid = lax.axis_index("s").astype(i32)
    w = sid * NC + cid
    nw = int(n_chunks) // NW
    rows_per_tile = n // NS

    li = lax.iota(i32, 16)
    lt2 = li < 2
    eq2 = li == 2
    one3 = jnp.where(li == 3, jnp.float32(1.0), jnp.float32(0.0))
    z16 = jnp.zeros((16,), f32)
    semg = (semg0, semg1)
    sems = (sems0, sems1)

    ix1 = pltpu.async_copy(src_hbm.at[pl.ds(w * nw, nw)], sidxs, semi)
    ix2 = pltpu.async_copy(dst_hbm.at[pl.ds(w * nw, nw)], didxs, semi)

    def zloop(_, r):
        stag[np.int32(0), r, pl.ds(0, 16)] = z16
        return r + np.int32(1)
    lax.fori_loop(np.int32(0), np.int32(C), zloop, jnp.int32(0), unroll=8)
    done = 0
    while done < rows_per_tile:
        step = min(C, rows_per_tile - done)
        pltpu.sync_copy(stag.at[np.int32(0), pl.ds(0, step)],
                        acc.at[pl.ds(sid * rows_per_tile + done, step)])
        done += step
    ix1.wait()
    ix2.wait()
    plsc.subcore_barrier()

    def fetch(b, j):
        b = np.int32(b)
        pltpu.async_copy(c16_hbm.at[sidxs.at[j]], srows.at[b], semg[b])
        pltpu.async_copy(c16_hbm.at[didxs.at[j]], drows.at[b], semg[b])

    def wait_rows(b):
        b = np.int32(b)
        pltpu.make_async_copy(c16_hbm.at[sidxs.at[np.int32(0)]], srows.at[b],
                              semg[b]).wait()
        pltpu.make_async_copy(c16_hbm.at[sidxs.at[np.int32(0)]], drows.at[b],
                              semg[b]).wait()

    def compute(b):
        b = np.int32(b)
        def edge_body(_, e):
            rel = srows[b, e, pl.ds(0, 16)] - drows[b, e, pl.ds(0, 16)]
            sq = rel * rel
            dv = _allsum16(sq, li) + jnp.float32(1e-12)
            distv = dv * _rsqrt16(dv)
            row = (jnp.where(lt2, rel, jnp.float32(0.0))
                   + jnp.where(eq2, distv, jnp.float32(0.0)) + one3)
            stag[b, e, pl.ds(0, 16)] = row
            return e + np.int32(1)
        lax.fori_loop(np.int32(0), np.int32(C), edge_body, jnp.int32(0),
                      unroll=4)

    def scat(b, j):
        b = np.int32(b)
        pltpu.async_copy(stag.at[b], acc.at[didxs.at[j]], sems[b], add=True)

    def wait_scat(b):
        b = np.int32(b)
        pltpu.make_async_copy(stag.at[b], acc.at[didxs.at[np.int32(0)]],
                              sems[b]).wait()

    fetch(0, jnp.int32(0))

    def pair_body(_, jj):
        c0 = jj * np.int32(2)
        fetch(1, c0 + np.int32(1))
        wait_rows(0)

        @pl.when(jj > np.int32(0))
        def _():
            wait_scat(0)
        compute(0)
        scat(0, c0)

        @pl.when(jj < np.int32((nw - 1) // 2))
        def _():
            fetch(0, c0 + np.int32(2))
        wait_rows(1)

        @pl.when(jj > np.int32(0))
        def _():
            wait_scat(1)
        compute(1)
        scat(1, c0 + np.int32(1))
        return jj + np.int32(1)
    lax.fori_loop(np.int32(0), np.int32(nw // 2), pair_body, jnp.int32(0))

    if nw % 2:
        wait_rows(0)
        wait_scat(0)
        compute(0)
        scat(0, jnp.int32(nw - 1))
        wait_scat(1)
        wait_scat(0)
    else:
        wait_scat(0)
        wait_scat(1)

    plsc.subcore_barrier()

    @pl.when(sid == 0)
    def _():
        pltpu.sync_copy(acc, out_hbm.at[cid])


def _geom_sc(coords16, src32, dst32, n, e):
    n_chunks = e // C
    mesh = plsc.VectorSubcoreMesh(core_axis_name="c", subcore_axis_name="s")
    kern = functools.partial(
        pl.kernel,
        out_type=jax.ShapeDtypeStruct((NC, n, 16), f32),
        mesh=mesh,
        compiler_params=pltpu.CompilerParams(needs_layout_passes=False, use_tc_tiling_on_sc=False),
        scratch_types=[
            pltpu.VMEM((n_chunks // NW, C), i32),
            pltpu.VMEM((n_chunks // NW, C), i32),
            pltpu.VMEM((2, C, 16), f32),
            pltpu.VMEM((2, C, 16), f32),
            pltpu.VMEM((2, C, 16), f32),
            pltpu.VMEM_SHARED((n, 16), f32),
            pltpu.SemaphoreType.DMA,
            pltpu.SemaphoreType.DMA,
            pltpu.SemaphoreType.DMA,
            pltpu.SemaphoreType.DMA,
            pltpu.SemaphoreType.DMA,
        ],
    )(functools.partial(_geom_body, n, n_chunks))
    return kern(coords16, src32.reshape(n_chunks, C),
                dst32.reshape(n_chunks, C))


def _attn_body(n, n_chunks, qnt_hbm, knn_hbm, xlo_hbm, xhi_hbm,
               src_hbm, dst_hbm, num_hbm, den_hbm,
               sidxs, didxs, qrows, krows, xrows, nstag, dstag,
               accn, accd, semi, semg0, semg1, sems0, sems1):
    cid = lax.axis_index("c").astype(i32)
    sid = lax.axis_index("s").astype(i32)
    nt = int(n_chunks) // NS
    rows_per_tile = n // NS

    li = lax.iota(i32, 16)
    oh0 = jnp.where(li == 0, jnp.float32(1.0), jnp.float32(0.0))
    z16 = jnp.zeros((16,), f32)
    semg = (semg0, semg1)
    sems = (sems0, sems1)

    ix1 = pltpu.async_copy(src_hbm.at[pl.ds(sid * nt, nt)], sidxs, semi)
    ix2 = pltpu.async_copy(dst_hbm.at[pl.ds(sid * nt, nt)], didxs, semi)

    lo8 = li < 8
    z0 = np.int32(0)

    def zloop(_, r):
        for cc in range(4):
            nstag[np.int32(0), r, pl.ds(16 * cc, 16)] = z16
        plsc.store_scatter(dstag.at[z0], [li * z0 + r, li], z16, mask=lo8)
        return r + np.int32(1)
    lax.fori_loop(np.int32(0), np.int32(C), zloop, jnp.int32(0), unroll=4)
    done = 0
    while done < rows_per_tile:
        step = min(C, rows_per_tile - done)
        off = sid * rows_per_tile + done
        pltpu.sync_copy(nstag.at[np.int32(0), pl.ds(0, step)],
                        accn.at[pl.ds(off, step)])
        pltpu.sync_copy(dstag.at[np.int32(0), pl.ds(0, step)],
                        accd.at[pl.ds(off, step)])
        done += step
    ix1.wait()
    ix2.wait()
    plsc.subcore_barrier()

    def fetch(b, j):
        b = np.int32(b)
        pltpu.async_copy(qnt_hbm.at[didxs.at[j]], qrows.at[b], semg[b])
        pltpu.async_copy(knn_hbm.at[sidxs.at[j]], krows.at[b], semg[b])

        @pl.when(cid == 0)
        def _():
            pltpu.async_copy(xlo_hbm.at[sidxs.at[j]], xrows.at[b], semg[b])

        @pl.when(cid == 1)
        def _():
            pltpu.async_copy(xhi_hbm.at[sidxs.at[j]], xrows.at[b], semg[b])

    def wait_rows(b):
        b = np.int32(b)
        for dst in (qrows, krows, xrows):
            pltpu.make_async_copy(qnt_hbm.at[sidxs.at[np.int32(0)]], dst.at[b],
                                  semg[b]).wait()

    def compute(b):
        b = np.int32(b)
        def edge_body(_, e):
            p = qrows[b, e, pl.ds(0, 16)] * krows[b, e, pl.ds(0, 16)]
            for cc in range(1, 4):
                p = p + (qrows[b, e, pl.ds(16 * cc, 16)]
                         * krows[b, e, pl.ds(16 * cc, 16)])
            av = jnp.exp(_allsum16(p, li))
            for cc in range(4):
                nstag[b, e, pl.ds(16 * cc, 16)] = (
                    av * xrows[b, e, pl.ds(16 * cc, 16)])
            plsc.store_scatter(dstag.at[b], [li * np.int32(0) + e, li],
                               av * oh0, mask=lo8)
            return e + np.int32(1)
        lax.fori_loop(np.int32(0), np.int32(C), edge_body, jnp.int32(0),
                      unroll=2)

    def scat(b, j):
        b = np.int32(b)
        pltpu.async_copy(nstag.at[b], accn.at[didxs.at[j]], sems[b], add=True)
        pltpu.async_copy(dstag.at[b], accd.at[didxs.at[j]], sems[b], add=True)

    def wait_scat(b):
        b = np.int32(b)
        pltpu.make_async_copy(nstag.at[b], accn.at[didxs.at[np.int32(0)]],
                              sems[b]).wait()
        pltpu.make_async_copy(dstag.at[b], accd.at[didxs.at[np.int32(0)]],
                              sems[b]).wait()

    fetch(0, jnp.int32(0))

    def pair_body(_, jj):
        c0 = jj * np.int32(2)
        fetch(1, c0 + np.int32(1))
        wait_rows(0)

        @pl.when(jj > np.int32(0))
        def _():
            wait_scat(0)
        compute(0)
        scat(0, c0)

        @pl.when(jj < np.int32((nt - 1) // 2))
        def _():
            fetch(0, c0 + np.int32(2))
        wait_rows(1)

        @pl.when(jj > np.int32(0))
        def _():
            wait_scat(1)
        compute(1)
        scat(1, c0 + np.int32(1))
        return jj + np.int32(1)
    lax.fori_loop(np.int32(0), np.int32(nt // 2), pair_body, jnp.int32(0))

    if nt % 2:
        wait_rows(0)
        wait_scat(0)
        compute(0)
        scat(0, jnp.int32(nt - 1))
        wait_scat(1)
        wait_scat(0)
    else:
        wait_scat(0)
        wait_scat(1)

    plsc.subcore_barrier()

    @pl.when(sid == 0)
    def _():
        pltpu.sync_copy(accn, num_hbm.at[cid])
        pltpu.sync_copy(accd, den_hbm.at[cid])


def _attn_sc(qnt, knn, xlo, xhi, src32, dst32, n, e):
    n_chunks = e // C
    mesh = plsc.VectorSubcoreMesh(core_axis_name="c", subcore_axis_name="s")
    kern = functools.partial(
        pl.kernel,
        out_type=(jax.ShapeDtypeStruct((NC, n, 64), f32),
                  jax.ShapeDtypeStruct((NC, n, 8), f32)),
        mesh=mesh,
        compiler_params=pltpu.CompilerParams(needs_layout_passes=False,
                                             use_tc_tiling_on_sc=False),
        scratch_types=[
            pltpu.VMEM((n_chunks // NS, C), i32),
            pltpu.VMEM((n_chunks // NS, C), i32),
            pltpu.VMEM((2, C, 64), f32),
            pltpu.VMEM((2, C, 64), f32),
            pltpu.VMEM((2, C, 64), f32),
            pltpu.VMEM((2, C, 64), f32),
            pltpu.VMEM((2, C, 8), f32),
            pltpu.VMEM_SHARED((n, 64), f32),
            pltpu.VMEM_SHARED((n, 8), f32),
            pltpu.SemaphoreType.DMA,
            pltpu.SemaphoreType.DMA,
            pltpu.SemaphoreType.DMA,
            pltpu.SemaphoreType.DMA,
            pltpu.SemaphoreType.DMA,
        ],
    )(functools.partial(_attn_body, n, n_chunks))
    return kern(qnt, knn, xlo, xhi, src32.reshape(n_chunks, C),
                dst32.reshape(n_chunks, C))


def _enc_body(gacc_ref, c2_ref, f_ref, F_ref, G8_ref, b_ref, plo_ref, phi_ref,
              x_ref, xlo_ref, xhi_ref, geom8_ref):
    acc = gacc_ref[0] + gacc_ref[1]
    deg = acc[:, 3:4]
    degc = jnp.maximum(deg, jnp.float32(1.0))
    mr = acc[:, 0:2] / degc
    md = acc[:, 2:3] / degc
    ld = jnp.log1p(deg)
    cc = c2_ref[...]
    geom8 = jnp.concatenate(
        [cc, mr, md, ld, jnp.zeros_like(cc)], axis=1)
    geom8_ref[...] = geom8
    h = (jnp.dot(f_ref[...], F_ref[...], preferred_element_type=f32)
         + jnp.dot(geom8, G8_ref[...], preferred_element_type=f32)
         + b_ref[...])
    x = jax.nn.gelu(h)
    x_ref[...] = x
    xlo_ref[...] = jnp.dot(x, plo_ref[...], preferred_element_type=f32)
    xhi_ref[...] = jnp.dot(x, phi_ref[...], preferred_element_type=f32)


def _qk_body(x_ref, wq_ref, wk_ref, tau_ref, qnt_ref, knn_ref):
    x = x_ref[...]
    q = jnp.dot(x, wq_ref[...], preferred_element_type=f32)
    k = jnp.dot(x, wk_ref[...], preferred_element_type=f32)
    qn = q / (jnp.sqrt(jnp.sum(q * q, axis=1, keepdims=True))
              + jnp.float32(1e-6))
    kn = k / (jnp.sqrt(jnp.sum(k * k, axis=1, keepdims=True))
              + jnp.float32(1e-6))
    qnt_ref[...] = qn * tau_ref[...]
    knn_ref[...] = kn


def _out_body(num_ref, den_ref, x_ref, g8_ref, wv_ref, B_ref, A_ref,
              Cg_ref, bo_ref, plo_ref, phi_ref,
              xo_ref, xolo_ref, xohi_ref):
    dn = (den_ref[0][:, 0:1] + den_ref[1][:, 0:1]) * jnp.float32(0.5)
    rden = jnp.float32(1.0) / (dn + jnp.float32(1e-9))
    m0 = num_ref[0] * rden
    m1 = num_ref[1] * rden
    wvb = jnp.dot(wv_ref[...], B_ref[...], preferred_element_type=f32)
    x = x_ref[...]
    h = (jnp.dot(x, A_ref[...], preferred_element_type=f32)
         + jnp.dot(m0, wvb[0:64, :], preferred_element_type=f32)
         + jnp.dot(m1, wvb[64:128, :], preferred_element_type=f32)
         + jnp.dot(g8_ref[...], Cg_ref[...], preferred_element_type=f32)
         + bo_ref[...])
    xo = x + jax.nn.gelu(h)
    xo_ref[...] = xo
    xolo_ref[...] = jnp.dot(xo, plo_ref[...], preferred_element_type=f32)
    xohi_ref[...] = jnp.dot(xo, phi_ref[...], preferred_element_type=f32)


def _row_spec(width):
    return pl.BlockSpec((BLK, width), lambda idx: (idx, _I0))


def _full_spec(shape):
    nd = len(shape)
    return pl.BlockSpec(shape, lambda idx: (_I0,) * nd)


def _acc_spec(width):
    return pl.BlockSpec((NC, BLK, width), lambda idx: (_I0, idx, _I0))


def kernel(coords, fields, multiscale_csr, W_enc, b_enc, Wq, Wk, Wv, Wo, bo,
           tau):
    n = coords.shape[0]
    e = multiscale_csr.shape[1]
    layers = Wq.shape[0]
    grid = (n // BLK,)

    src32 = multiscale_csr[0].astype(i32)
    dst32 = multiscale_csr[1].astype(i32)
    coords16 = jnp.zeros((n, 16), f32).at[:, 0:2].set(coords.astype(f32))

    gacc = _geom_sc(coords16, src32, dst32, n, e)

    Fw = W_enc[:128].astype(f32)
    G8 = jnp.zeros((8, 128), f32).at[0:6, :].set(W_enc[128:134].astype(f32))
    b_row = b_enc.astype(f32).reshape(1, 128)
    eye = jnp.eye(128, dtype=f32)
    plo = eye[:, 0:64]
    phi = eye[:, 64:128]

    x, xlo, xhi, geom8 = pl.pallas_call(
        _enc_body,
        grid=grid,
        in_specs=[
            _acc_spec(16),
            _row_spec(2),
            _row_spec(128),
            _full_spec((128, 128)),
            _full_spec((8, 128)),
            _full_spec((1, 128)),
            _full_spec((128, 64)),
            _full_spec((128, 64)),
        ],
        out_specs=(_row_spec(128), _row_spec(64), _row_spec(64),
                   _row_spec(8)),
        out_shape=(jax.ShapeDtypeStruct((n, 128), f32),
                   jax.ShapeDtypeStruct((n, 64), f32),
                   jax.ShapeDtypeStruct((n, 64), f32),
                   jax.ShapeDtypeStruct((n, 8), f32)),
    )(gacc, coords.astype(f32), fields.astype(f32), Fw, G8, b_row, plo, phi)

    for l in range(layers):
        tau_row = jnp.full((1, 64), tau[l], f32)
        qnt, knn = pl.pallas_call(
            _qk_body,
            grid=grid,
            in_specs=[
                _row_spec(128),
                _full_spec((128, 64)),
                _full_spec((128, 64)),
                _full_spec((1, 64)),
            ],
            out_specs=(_row_spec(64), _row_spec(64)),
            out_shape=(jax.ShapeDtypeStruct((n, 64), f32),
                       jax.ShapeDtypeStruct((n, 64), f32)),
        )(x, Wq[l].astype(f32), Wk[l].astype(f32), tau_row)

        num, den = _attn_sc(qnt, knn, xlo, xhi, src32, dst32, n, e)

        x, xlo, xhi = pl.pallas_call(
            _out_body,
            grid=grid,
            in_specs=[
                _acc_spec(64),
                _acc_spec(8),
                _row_spec(128),
                _row_spec(8),
                _full_spec((128, 128)),
                _full_spec((128, 128)),
                _full_spec((128, 128)),
                _full_spec((8, 128)),
                _full_spec((1, 128)),
                _full_spec((128, 64)),
                _full_spec((128, 64)),
            ],
            out_specs=(_row_spec(128), _row_spec(64), _row_spec(64)),
            out_shape=(jax.ShapeDtypeStruct((n, 128), f32),
                       jax.ShapeDtypeStruct((n, 64), f32),
                       jax.ShapeDtypeStruct((n, 64), f32)),
        )(num, den, x, geom8, Wv[l].astype(f32),
          Wo[l, 128:256].astype(f32), Wo[l, 0:128].astype(f32),
          jnp.zeros((8, 128), f32).at[0:6, :].set(Wo[l, 256:262].astype(f32)),
          bo[l].astype(f32).reshape(1, 128), plo, phi)
    return x


# speedup vs baseline: 8.2164x; 1.1296x over previous
import functools
import numpy as np
import jax
import jax.numpy as jnp
from jax import lax
from jax.experimental import pallas as pl
from jax.experimental.pallas import tpu as pltpu
from jax.experimental.pallas import tpu_sc as plsc

NC = 2
NS = 16
NW = NC * NS
C = 80
BLK = 1000

_I0 = np.int32(0)
f32 = jnp.float32
i32 = jnp.int32


def _rsqrt16(d):
    i = plsc.bitcast(d, i32)
    i = jnp.int32(0x5F3759DF) - lax.shift_right_arithmetic(i, jnp.int32(1))
    y = plsc.bitcast(i, f32)
    for _ in range(3):
        y = y * (jnp.float32(1.5) - jnp.float32(0.5) * d * y * y)
    return y


def _allsum16(p, li):
    for k in (8, 4, 2, 1):
        idx = lax.bitwise_xor(li, jnp.int32(k))
        perm = lax.gather(
            p, idx[:, None],
            lax.GatherDimensionNumbers(offset_dims=(),
                                       collapsed_slice_dims=(0,),
                                       start_index_map=(0,)),
            slice_sizes=(1,),
            mode=lax.GatherScatterMode.PROMISE_IN_BOUNDS)
        p = p + perm
    return p


def _worker_chunks(n_chunks):
    w = lax.axis_index("s") * NC + lax.axis_index("c")
    base = int(n_chunks) // NW
    extra = int(n_chunks) - base * NW
    nw_chunks = jnp.where(w < np.int32(extra),
                          np.int32(base + 1), np.int32(base))
    return w.astype(i32), nw_chunks


def _geom_body(n, n_chunks, cx_hbm, cy_hbm, src_hbm, dst_hbm, out_hbm,
               sidxs, didxs, cxv, cyv, stag, acc, semi, sems0, sems1):
    cid = lax.axis_index("c").astype(i32)
    sid = lax.axis_index("s").astype(i32)
    w = sid * NC + cid
    nw = int(n_chunks) // NW
    rows_per_tile = n // NS

    li = lax.iota(i32, 16)
    z16 = jnp.zeros((16,), f32)
    one16 = jnp.full((16,), 1.0, f32)
    col0 = jnp.full((16,), 0, i32)
    col1 = jnp.full((16,), 1, i32)
    col2 = jnp.full((16,), 2, i32)
    col3 = jnp.full((16,), 3, i32)
    sems = (sems0, sems1)

    ix1 = pltpu.async_copy(src_hbm.at[pl.ds(w * nw, nw)], sidxs, semi)
    ix2 = pltpu.async_copy(dst_hbm.at[pl.ds(w * nw, nw)], didxs, semi)
    ix3 = pltpu.async_copy(cx_hbm, cxv, semi)
    ix4 = pltpu.async_copy(cy_hbm, cyv, semi)

    def zloop(_, r):
        stag[np.int32(0), r, pl.ds(0, 16)] = z16
        stag[np.int32(1), r, pl.ds(0, 16)] = z16
        return r + np.int32(1)
    lax.fori_loop(np.int32(0), np.int32(C), zloop, jnp.int32(0), unroll=8)
    done = 0
    while done < rows_per_tile:
        step = min(C, rows_per_tile - done)
        pltpu.sync_copy(stag.at[np.int32(0), pl.ds(0, step)],
                        acc.at[pl.ds(sid * rows_per_tile + done, step)])
        done += step
    ix1.wait()
    ix2.wait()
    ix3.wait()
    ix4.wait()
    plsc.subcore_barrier()

    def compute(b, j):
        b = np.int32(b)
        for g in range(C // 16):
            si = sidxs[j, pl.ds(g * 16, 16)]
            di = didxs[j, pl.ds(g * 16, 16)]
            relx = plsc.load_gather(cxv, [si]) - plsc.load_gather(cxv, [di])
            rely = plsc.load_gather(cyv, [si]) - plsc.load_gather(cyv, [di])
            d2 = relx * relx + rely * rely + jnp.float32(1e-12)
            dist = d2 * _rsqrt16(d2)
            rows = li + np.int32(g * 16)
            plsc.store_scatter(stag.at[b], [rows, col0], relx)
            plsc.store_scatter(stag.at[b], [rows, col1], rely)
            plsc.store_scatter(stag.at[b], [rows, col2], dist)
            plsc.store_scatter(stag.at[b], [rows, col3], one16)

    def scat(b, j):
        b = np.int32(b)
        pltpu.async_copy(stag.at[b], acc.at[didxs.at[j]], sems[b], add=True)

    def wait_scat(b):
        b = np.int32(b)
        pltpu.make_async_copy(stag.at[b], acc.at[didxs.at[np.int32(0)]],
                              sems[b]).wait()

    def pair_body(_, jj):
        c0 = jj * np.int32(2)

        @pl.when(jj > np.int32(0))
        def _():
            wait_scat(0)
        compute(0, c0)
        scat(0, c0)

        @pl.when(jj > np.int32(0))
        def _():
            wait_scat(1)
        compute(1, c0 + np.int32(1))
        scat(1, c0 + np.int32(1))
        return jj + np.int32(1)
    lax.fori_loop(np.int32(0), np.int32(nw // 2), pair_body, jnp.int32(0))

    if nw % 2:
        wait_scat(0)
        compute(0, jnp.int32(nw - 1))
        scat(0, jnp.int32(nw - 1))
        wait_scat(1)
        wait_scat(0)
    else:
        wait_scat(0)
        wait_scat(1)

    plsc.subcore_barrier()

    @pl.when(sid == 0)
    def _():
        pltpu.sync_copy(acc, out_hbm.at[cid])


def _geom_sc(cx, cy, src32, dst32, n, e):
    n_chunks = e // C
    mesh = plsc.VectorSubcoreMesh(core_axis_name="c", subcore_axis_name="s")
    kern = functools.partial(
        pl.kernel,
        out_type=jax.ShapeDtypeStruct((NC, n, 16), f32),
        mesh=mesh,
        compiler_params=pltpu.CompilerParams(needs_layout_passes=False,
                                             use_tc_tiling_on_sc=False),
        scratch_types=[
            pltpu.VMEM((n_chunks // NW, C), i32),
            pltpu.VMEM((n_chunks // NW, C), i32),
            pltpu.VMEM((n,), f32),
            pltpu.VMEM((n,), f32),
            pltpu.VMEM((2, C, 16), f32),
            pltpu.VMEM_SHARED((n, 16), f32),
            pltpu.SemaphoreType.DMA,
            pltpu.SemaphoreType.DMA,
            pltpu.SemaphoreType.DMA,
        ],
    )(functools.partial(_geom_body, n, n_chunks))
    return kern(cx, cy, src32.reshape(n_chunks, C),
                dst32.reshape(n_chunks, C))


def _attn_body(n, n_chunks, qnt_hbm, kxlo_hbm, kxhi_hbm,
               src_hbm, dst_hbm, num_hbm, den_hbm,
               sidxs, didxs, qrows, kxrows, nstag, dstag,
               accn, accd, semi, semg0, semg1, sems0, sems1):
    cid = lax.axis_index("c").astype(i32)
    sid = lax.axis_index("s").astype(i32)
    nt = int(n_chunks) // NS
    rows_per_tile = n // NS

    li = lax.iota(i32, 16)
    oh0 = jnp.where(li == 0, jnp.float32(1.0), jnp.float32(0.0))
    z16 = jnp.zeros((16,), f32)
    semg = (semg0, semg1)
    sems = (sems0, sems1)

    ix1 = pltpu.async_copy(src_hbm.at[pl.ds(sid * nt, nt)], sidxs, semi)
    ix2 = pltpu.async_copy(dst_hbm.at[pl.ds(sid * nt, nt)], didxs, semi)

    lo8 = li < 8
    z0 = np.int32(0)

    def zloop(_, r):
        for cc in range(4):
            nstag[np.int32(0), r, pl.ds(16 * cc, 16)] = z16
        plsc.store_scatter(dstag.at[z0], [li * z0 + r, li], z16, mask=lo8)
        return r + np.int32(1)
    lax.fori_loop(np.int32(0), np.int32(C), zloop, jnp.int32(0), unroll=4)
    done = 0
    while done < rows_per_tile:
        step = min(C, rows_per_tile - done)
        off = sid * rows_per_tile + done
        pltpu.sync_copy(nstag.at[np.int32(0), pl.ds(0, step)],
                        accn.at[pl.ds(off, step)])
        pltpu.sync_copy(dstag.at[np.int32(0), pl.ds(0, step)],
                        accd.at[pl.ds(off, step)])
        done += step
    ix1.wait()
    ix2.wait()
    plsc.subcore_barrier()

    def fetch(b, j):
        b = np.int32(b)
        pltpu.async_copy(qnt_hbm.at[didxs.at[j]], qrows.at[b], semg[b])

        @pl.when(cid == 0)
        def _():
            pltpu.async_copy(kxlo_hbm.at[sidxs.at[j]], kxrows.at[b], semg[b])

        @pl.when(cid == 1)
        def _():
            pltpu.async_copy(kxhi_hbm.at[sidxs.at[j]], kxrows.at[b], semg[b])

    def wait_rows(b):
        b = np.int32(b)
        pltpu.make_async_copy(qnt_hbm.at[sidxs.at[np.int32(0)]], qrows.at[b],
                              semg[b]).wait()
        pltpu.make_async_copy(kxlo_hbm.at[sidxs.at[np.int32(0)]],
                              kxrows.at[b], semg[b]).wait()

    def compute(b):
        b = np.int32(b)
        def edge_body(_, e):
            p = qrows[b, e, pl.ds(0, 16)] * kxrows[b, e, pl.ds(0, 16)]
            for cc in range(1, 4):
                p = p + (qrows[b, e, pl.ds(16 * cc, 16)]
                         * kxrows[b, e, pl.ds(16 * cc, 16)])
            av = jnp.exp(_allsum16(p, li))
            for cc in range(4):
                nstag[b, e, pl.ds(16 * cc, 16)] = (
                    av * kxrows[b, e, pl.ds(64 + 16 * cc, 16)])
            plsc.store_scatter(dstag.at[b], [li * np.int32(0) + e, li],
                               av * oh0, mask=lo8)
            return e + np.int32(1)
        lax.fori_loop(np.int32(0), np.int32(C), edge_body, jnp.int32(0),
                      unroll=2)

    def scat(b, j):
        b = np.int32(b)
        pltpu.async_copy(nstag.at[b], accn.at[didxs.at[j]], sems[b], add=True)
        pltpu.async_copy(dstag.at[b], accd.at[didxs.at[j]], sems[b], add=True)

    def wait_scat(b):
        b = np.int32(b)
        pltpu.make_async_copy(nstag.at[b], accn.at[didxs.at[np.int32(0)]],
                              sems[b]).wait()
        pltpu.make_async_copy(dstag.at[b], accd.at[didxs.at[np.int32(0)]],
                              sems[b]).wait()

    fetch(0, jnp.int32(0))

    def pair_body(_, jj):
        c0 = jj * np.int32(2)
        fetch(1, c0 + np.int32(1))
        wait_rows(0)

        @pl.when(jj > np.int32(0))
        def _():
            wait_scat(0)
        compute(0)
        scat(0, c0)

        @pl.when(jj < np.int32((nt - 1) // 2))
        def _():
            fetch(0, c0 + np.int32(2))
        wait_rows(1)

        @pl.when(jj > np.int32(0))
        def _():
            wait_scat(1)
        compute(1)
        scat(1, c0 + np.int32(1))
        return jj + np.int32(1)
    lax.fori_loop(np.int32(0), np.int32(nt // 2), pair_body, jnp.int32(0))

    if nt % 2:
        wait_rows(0)
        wait_scat(0)
        compute(0)
        scat(0, jnp.int32(nt - 1))
        wait_scat(1)
        wait_scat(0)
    else:
        wait_scat(0)
        wait_scat(1)

    plsc.subcore_barrier()

    @pl.when(sid == 0)
    def _():
        pltpu.sync_copy(accn, num_hbm.at[cid])
        pltpu.sync_copy(accd, den_hbm.at[cid])


def _attn_sc(qnt, kxlo, kxhi, src32, dst32, n, e):
    n_chunks = e // C
    mesh = plsc.VectorSubcoreMesh(core_axis_name="c", subcore_axis_name="s")
    kern = functools.partial(
        pl.kernel,
        out_type=(jax.ShapeDtypeStruct((NC, n, 64), f32),
                  jax.ShapeDtypeStruct((NC, n, 8), f32)),
        mesh=mesh,
        compiler_params=pltpu.CompilerParams(needs_layout_passes=False,
                                             use_tc_tiling_on_sc=False),
        scratch_types=[
            pltpu.VMEM((n_chunks // NS, C), i32),
            pltpu.VMEM((n_chunks // NS, C), i32),
            pltpu.VMEM((2, C, 64), f32),
            pltpu.VMEM((2, C, 128), f32),
            pltpu.VMEM((2, C, 64), f32),
            pltpu.VMEM((2, C, 8), f32),
            pltpu.VMEM_SHARED((n, 64), f32),
            pltpu.VMEM_SHARED((n, 8), f32),
            pltpu.SemaphoreType.DMA,
            pltpu.SemaphoreType.DMA,
            pltpu.SemaphoreType.DMA,
            pltpu.SemaphoreType.DMA,
            pltpu.SemaphoreType.DMA,
        ],
    )(functools.partial(_attn_body, n, n_chunks))
    return kern(qnt, kxlo, kxhi, src32.reshape(n_chunks, C),
                dst32.reshape(n_chunks, C))


def _enc_body(gacc_ref, c2_ref, f_ref, F_ref, G8_ref, b_ref, plo_ref, phi_ref,
              x_ref, xlo_ref, xhi_ref, geom8_ref):
    acc = gacc_ref[0] + gacc_ref[1]
    deg = acc[:, 3:4]
    degc = jnp.maximum(deg, jnp.float32(1.0))
    mr = acc[:, 0:2] / degc
    md = acc[:, 2:3] / degc
    ld = jnp.log1p(deg)
    cc = c2_ref[...]
    geom8 = jnp.concatenate(
        [cc, mr, md, ld, jnp.zeros_like(cc)], axis=1)
    geom8_ref[...] = geom8
    h = (jnp.dot(f_ref[...], F_ref[...], preferred_element_type=f32)
         + jnp.dot(geom8, G8_ref[...], preferred_element_type=f32)
         + b_ref[...])
    x = jax.nn.gelu(h)
    x_ref[...] = x
    xlo_ref[...] = jnp.dot(x, plo_ref[...], preferred_element_type=f32)
    xhi_ref[...] = jnp.dot(x, phi_ref[...], preferred_element_type=f32)


def _qk_body(x_ref, xlo_ref, xhi_ref, wq_ref, wk_ref, tau_ref, slo_ref,
             shi_ref, qnt_ref, kxlo_ref, kxhi_ref):
    x = x_ref[...]
    q = jnp.dot(x, wq_ref[...], preferred_element_type=f32)
    k = jnp.dot(x, wk_ref[...], preferred_element_type=f32)
    qn = q / (jnp.sqrt(jnp.sum(q * q, axis=1, keepdims=True))
              + jnp.float32(1e-6))
    kn = k / (jnp.sqrt(jnp.sum(k * k, axis=1, keepdims=True))
              + jnp.float32(1e-6))
    qnt_ref[...] = qn * tau_ref[...]
    kh = jnp.dot(kn, slo_ref[...], preferred_element_type=f32)
    kxlo_ref[...] = kh + jnp.dot(xlo_ref[...], shi_ref[...],
                                 preferred_element_type=f32)
    kxhi_ref[...] = kh + jnp.dot(xhi_ref[...], shi_ref[...],
                                 preferred_element_type=f32)


def _out_body(num_ref, den_ref, x_ref, g8_ref, wv_ref, B_ref, A_ref,
              Cg_ref, bo_ref, plo_ref, phi_ref,
              xo_ref, xolo_ref, xohi_ref):
    dn = (den_ref[0][:, 0:1] + den_ref[1][:, 0:1]) * jnp.float32(0.5)
    rden = jnp.float32(1.0) / (dn + jnp.float32(1e-9))
    m0 = num_ref[0] * rden
    m1 = num_ref[1] * rden
    wvb = jnp.dot(wv_ref[...], B_ref[...], preferred_element_type=f32)
    x = x_ref[...]
    h = (jnp.dot(x, A_ref[...], preferred_element_type=f32)
         + jnp.dot(m0, wvb[0:64, :], preferred_element_type=f32)
         + jnp.dot(m1, wvb[64:128, :], preferred_element_type=f32)
         + jnp.dot(g8_ref[...], Cg_ref[...], preferred_element_type=f32)
         + bo_ref[...])
    xo = x + jax.nn.gelu(h)
    xo_ref[...] = xo
    xolo_ref[...] = jnp.dot(xo, plo_ref[...], preferred_element_type=f32)
    xohi_ref[...] = jnp.dot(xo, phi_ref[...], preferred_element_type=f32)


def _row_spec(width):
    return pl.BlockSpec((BLK, width), lambda idx: (idx, _I0))


def _full_spec(shape):
    nd = len(shape)
    return pl.BlockSpec(shape, lambda idx: (_I0,) * nd)


def _acc_spec(width):
    return pl.BlockSpec((NC, BLK, width), lambda idx: (_I0, idx, _I0))


def kernel(coords, fields, multiscale_csr, W_enc, b_enc, Wq, Wk, Wv, Wo, bo,
           tau):
    n = coords.shape[0]
    e = multiscale_csr.shape[1]
    layers = Wq.shape[0]
    grid = (n // BLK,)

    src32 = multiscale_csr[0].astype(i32)
    dst32 = multiscale_csr[1].astype(i32)
    cxy = coords.astype(f32)

    gacc = _geom_sc(cxy[:, 0], cxy[:, 1], src32, dst32, n, e)

    Fw = W_enc[:128].astype(f32)
    G8 = jnp.zeros((8, 128), f32).at[0:6, :].set(W_enc[128:134].astype(f32))
    b_row = b_enc.astype(f32).reshape(1, 128)
    eye = jnp.eye(128, dtype=f32)
    plo = eye[:, 0:64]
    phi = eye[:, 64:128]
    slo = eye[0:64, :]
    shi = eye[64:128, :]

    x, xlo, xhi, geom8 = pl.pallas_call(
        _enc_body,
        grid=grid,
        in_specs=[
            _acc_spec(16),
            _row_spec(2),
            _row_spec(128),
            _full_spec((128, 128)),
            _full_spec((8, 128)),
            _full_spec((1, 128)),
            _full_spec((128, 64)),
            _full_spec((128, 64)),
        ],
        out_specs=(_row_spec(128), _row_spec(64), _row_spec(64),
                   _row_spec(8)),
        out_shape=(jax.ShapeDtypeStruct((n, 128), f32),
                   jax.ShapeDtypeStruct((n, 64), f32),
                   jax.ShapeDtypeStruct((n, 64), f32),
                   jax.ShapeDtypeStruct((n, 8), f32)),
    )(gacc, coords.astype(f32), fields.astype(f32), Fw, G8, b_row, plo, phi)

    for l in range(layers):
        tau_row = jnp.full((1, 64), tau[l], f32)
        qnt, kxlo, kxhi = pl.pallas_call(
            _qk_body,
            grid=grid,
            in_specs=[
                _row_spec(128),
                _row_spec(64),
                _row_spec(64),
                _full_spec((128, 64)),
                _full_spec((128, 64)),
                _full_spec((1, 64)),
                _full_spec((64, 128)),
                _full_spec((64, 128)),
            ],
            out_specs=(_row_spec(64), _row_spec(128), _row_spec(128)),
            out_shape=(jax.ShapeDtypeStruct((n, 64), f32),
                       jax.ShapeDtypeStruct((n, 128), f32),
                       jax.ShapeDtypeStruct((n, 128), f32)),
        )(x, xlo, xhi, Wq[l].astype(f32), Wk[l].astype(f32), tau_row,
          slo, shi)

        num, den = _attn_sc(qnt, kxlo, kxhi, src32, dst32, n, e)

        x, xlo, xhi = pl.pallas_call(
            _out_body,
            grid=grid,
            in_specs=[
                _acc_spec(64),
                _acc_spec(8),
                _row_spec(128),
                _row_spec(8),
                _full_spec((128, 128)),
                _full_spec((128, 128)),
                _full_spec((128, 128)),
                _full_spec((8, 128)),
                _full_spec((1, 128)),
                _full_spec((128, 64)),
                _full_spec((128, 64)),
            ],
            out_specs=(_row_spec(128), _row_spec(64), _row_spec(64)),
            out_shape=(jax.ShapeDtypeStruct((n, 128), f32),
                       jax.ShapeDtypeStruct((n, 64), f32),
                       jax.ShapeDtypeStruct((n, 64), f32)),
        )(num, den, x, geom8, Wv[l].astype(f32),
          Wo[l, 128:256].astype(f32), Wo[l, 0:128].astype(f32),
          jnp.zeros((8, 128), f32).at[0:6, :].set(Wo[l, 256:262].astype(f32)),
          bo[l].astype(f32).reshape(1, 128), plo, phi)
    return x
